# Initial kernel scaffold; baseline (speedup 1.0000x reference)
import jax
import jax.numpy as jnp
from jax.experimental import pallas as pl


def kernel(x, W, expert_bias):
    raise NotImplementedError("write your pallas kernel here")


# speedup vs baseline: 5.0953x; 5.0953x over previous
import functools

import jax
import jax.numpy as jnp
from jax.experimental import pallas as pl
from jax.experimental.pallas import tpu as pltpu

_D_MODEL = 768
_N_EXPERTS = 64
_TOP_K = 8
_AUX_COEF = 0.01
_BLOCK_M = 1024


def _router_block(x_ref, w_ref, b_ref, gates_ref, idx_ref, aux_ref,
                  f_acc, p_acc, *, n_tokens):
    i = pl.program_id(0)
    nsteps = pl.num_programs(0)

    x = x_ref[...]
    w = w_ref[...]
    logits = jax.lax.dot_general(
        x, w, (((1,), (1,)), ((), ())),
        preferred_element_type=jnp.float32,
    )
    biased = logits + b_ref[...]

    iota = jax.lax.broadcasted_iota(jnp.int32, logits.shape, 1)
    cur = biased
    sel_vals = []
    sel_idx = []
    for _ in range(_TOP_K):
        mx = jnp.max(cur, axis=1, keepdims=True)
        idxk = jnp.min(
            jnp.where(cur == mx, iota, _N_EXPERTS), axis=1, keepdims=True)
        onehot = iota == idxk
        sel_vals.append(
            jnp.sum(jnp.where(onehot, logits, 0.0), axis=1, keepdims=True))
        sel_idx.append(idxk)
        cur = jnp.where(onehot, -jnp.inf, cur)

    vals = jnp.concatenate(sel_vals, axis=1)
    idxs = jnp.concatenate(sel_idx, axis=1)
    m = jnp.max(vals, axis=1, keepdims=True)
    e = jnp.exp(vals - m)
    g = e / jnp.sum(e, axis=1, keepdims=True)

    gates = jnp.zeros_like(logits)
    for k in range(_TOP_K):
        gates += jnp.where(iota == sel_idx[k], g[:, k:k + 1], 0.0)
    gates_ref[...] = gates
    idx_ref[...] = idxs

    f_part = jnp.sum((gates > 0).astype(jnp.float32), axis=0, keepdims=True)
    ml = jnp.max(logits, axis=1, keepdims=True)
    el = jnp.exp(logits - ml)
    p = el / jnp.sum(el, axis=1, keepdims=True)
    p_part = jnp.sum(p, axis=0, keepdims=True)

    @pl.when(i == 0)
    def _init():
        f_acc[...] = jnp.zeros_like(f_acc)
        p_acc[...] = jnp.zeros_like(p_acc)
        aux_ref[...] = jnp.zeros_like(aux_ref)

    f_acc[...] += f_part
    p_acc[...] += p_part

    @pl.when(i == nsteps - 1)
    def _finish():
        scale = _AUX_COEF * _N_EXPERTS / (float(n_tokens) * float(n_tokens))
        aux_ref[...] = (scale * jnp.sum(f_acc[...] * p_acc[...]))[None, None]


def kernel(x, W, expert_bias):
    n_tokens, d_model = x.shape
    n_experts = W.shape[0]
    bm = _BLOCK_M
    grid = (n_tokens // bm,)

    gates, idxs, aux = pl.pallas_call(
        functools.partial(_router_block, n_tokens=n_tokens),
        grid=grid,
        in_specs=[
            pl.BlockSpec((bm, d_model), lambda i: (i, 0)),
            pl.BlockSpec((n_experts, d_model), lambda i: (0, 0)),
            pl.BlockSpec((1, n_experts), lambda i: (0, 0)),
        ],
        out_specs=[
            pl.BlockSpec((bm, n_experts), lambda i: (i, 0)),
            pl.BlockSpec((bm, _TOP_K), lambda i: (i, 0)),
            pl.BlockSpec((1, 1), lambda i: (0, 0)),
        ],
        out_shape=[
            jax.ShapeDtypeStruct((n_tokens, n_experts), jnp.float32),
            jax.ShapeDtypeStruct((n_tokens, _TOP_K), jnp.int32),
            jax.ShapeDtypeStruct((1, 1), jnp.float32),
        ],
        scratch_shapes=[
            pltpu.VMEM((1, n_experts), jnp.float32),
            pltpu.VMEM((1, n_experts), jnp.float32),
        ],
    )(x, W, expert_bias.reshape(1, n_experts))
    return gates, idxs, aux[0, 0]


# speedup vs baseline: 9.7316x; 1.9099x over previous
import functools

import jax
import jax.numpy as jnp
from jax.experimental import pallas as pl
from jax.experimental.pallas import tpu as pltpu

_D_MODEL = 768
_N_EXPERTS = 64
_TOP_K = 8
_AUX_COEF = 0.01
_BLOCK_M = 1024


def _router_block(x_ref, w_ref, b_ref, gates_ref, idx_ref, aux_ref,
                  f_acc, p_acc, *, n_tokens):
    i = pl.program_id(0)
    nsteps = pl.num_programs(0)

    x = x_ref[...]
    w = w_ref[...]
    logits = jax.lax.dot_general(
        x, w, (((1,), (1,)), ((), ())),
        preferred_element_type=jnp.float32,
    )
    biased = logits + b_ref[...]

    rev = jnp.float32(_N_EXPERTS) - jax.lax.broadcasted_iota(
        jnp.int32, logits.shape, 1).astype(jnp.float32)
    cur = biased
    sel_mask = jnp.zeros_like(logits)
    sel_idx_f = []
    for _ in range(_TOP_K):
        mx = jnp.max(cur, axis=1, keepdims=True)
        r = jnp.max(
            jnp.where(cur == mx, rev, 0.0), axis=1, keepdims=True)
        onehot = rev == r
        sel_mask += onehot.astype(jnp.float32)
        sel_idx_f.append(jnp.float32(_N_EXPERTS) - r)
        cur = jnp.where(onehot, -jnp.inf, cur)

    idxs = jnp.concatenate(sel_idx_f, axis=1).astype(jnp.int32)

    masked = jnp.where(sel_mask > 0, logits, -jnp.inf)
    m = jnp.max(masked, axis=1, keepdims=True)
    e = jnp.exp(masked - m)
    gates = e / jnp.sum(e, axis=1, keepdims=True)
    gates_ref[...] = gates
    idx_ref[...] = idxs

    f_part = jnp.sum((gates > 0).astype(jnp.float32), axis=0, keepdims=True)
    ml = jnp.max(logits, axis=1, keepdims=True)
    el = jnp.exp(logits - ml)
    p = el / jnp.sum(el, axis=1, keepdims=True)
    p_part = jnp.sum(p, axis=0, keepdims=True)

    @pl.when(i == 0)
    def _init():
        f_acc[...] = jnp.zeros_like(f_acc)
        p_acc[...] = jnp.zeros_like(p_acc)
        aux_ref[...] = jnp.zeros_like(aux_ref)

    f_acc[...] += f_part
    p_acc[...] += p_part

    @pl.when(i == nsteps - 1)
    def _finish():
        scale = _AUX_COEF * _N_EXPERTS / (float(n_tokens) * float(n_tokens))
        aux_ref[...] = (scale * jnp.sum(f_acc[...] * p_acc[...]))[None, None]


def kernel(x, W, expert_bias):
    n_tokens, d_model = x.shape
    n_experts = W.shape[0]
    bm = _BLOCK_M
    grid = (n_tokens // bm,)

    gates, idxs, aux = pl.pallas_call(
        functools.partial(_router_block, n_tokens=n_tokens),
        grid=grid,
        in_specs=[
            pl.BlockSpec((bm, d_model), lambda i: (i, 0)),
            pl.BlockSpec((n_experts, d_model), lambda i: (0, 0)),
            pl.BlockSpec((1, n_experts), lambda i: (0, 0)),
        ],
        out_specs=[
            pl.BlockSpec((bm, n_experts), lambda i: (i, 0)),
            pl.BlockSpec((bm, _TOP_K), lambda i: (i, 0)),
            pl.BlockSpec((1, 1), lambda i: (0, 0)),
        ],
        out_shape=[
            jax.ShapeDtypeStruct((n_tokens, n_experts), jnp.float32),
            jax.ShapeDtypeStruct((n_tokens, _TOP_K), jnp.int32),
            jax.ShapeDtypeStruct((1, 1), jnp.float32),
        ],
        scratch_shapes=[
            pltpu.VMEM((1, n_experts), jnp.float32),
            pltpu.VMEM((1, n_experts), jnp.float32),
        ],
    )(x, W, expert_bias.reshape(1, n_experts))
    return gates, idxs, aux[0, 0]


# speedup vs baseline: 13.5068x; 1.3879x over previous
import functools

import jax
import jax.numpy as jnp
from jax.experimental import pallas as pl
from jax.experimental.pallas import tpu as pltpu

_D_MODEL = 768
_N_EXPERTS = 64
_TOP_K = 8
_AUX_COEF = 0.01
_BLOCK_M = 1024


def _router_block(x_ref, w_ref, b_ref, gates_ref, idx_ref, aux_ref,
                  f_acc, p_acc, *, n_tokens):
    i = pl.program_id(0)
    nsteps = pl.num_programs(0)

    logits = jax.lax.dot_general(
        w_ref[...], x_ref[...], (((1,), (1,)), ((), ())),
        preferred_element_type=jnp.float32,
    )
    biased = logits + b_ref[...]

    rev = jnp.float32(_N_EXPERTS) - jax.lax.broadcasted_iota(
        jnp.int32, logits.shape, 0).astype(jnp.float32)
    cur = biased
    sel_mask = jnp.zeros_like(logits)
    sel_idx_f = []
    for _ in range(_TOP_K):
        mx = jnp.max(cur, axis=0, keepdims=True)
        r = jnp.max(
            jnp.where(cur == mx, rev, 0.0), axis=0, keepdims=True)
        onehot = rev == r
        sel_mask += onehot.astype(jnp.float32)
        sel_idx_f.append(jnp.float32(_N_EXPERTS) - r)
        cur = jnp.where(onehot, -jnp.inf, cur)

    idxs = jnp.concatenate(sel_idx_f, axis=0).astype(jnp.int32)

    masked = jnp.where(sel_mask > 0, logits, -jnp.inf)
    m = jnp.max(masked, axis=0, keepdims=True)
    e = jnp.exp(masked - m)
    gates = e / jnp.sum(e, axis=0, keepdims=True)

    gates_ref[...] = gates.T
    idx_ref[...] = idxs.T

    f_part = jnp.sum((gates > 0).astype(jnp.float32), axis=1, keepdims=True)
    ml = jnp.max(logits, axis=0, keepdims=True)
    el = jnp.exp(logits - ml)
    p = el / jnp.sum(el, axis=0, keepdims=True)
    p_part = jnp.sum(p, axis=1, keepdims=True)

    @pl.when(i == 0)
    def _init():
        f_acc[...] = jnp.zeros_like(f_acc)
        p_acc[...] = jnp.zeros_like(p_acc)
        aux_ref[...] = jnp.zeros_like(aux_ref)

    f_acc[...] += f_part
    p_acc[...] += p_part

    @pl.when(i == nsteps - 1)
    def _finish():
        scale = _AUX_COEF * _N_EXPERTS / (float(n_tokens) * float(n_tokens))
        aux_ref[...] = (scale * jnp.sum(f_acc[...] * p_acc[...]))[None, None]


def kernel(x, W, expert_bias):
    n_tokens, d_model = x.shape
    n_experts = W.shape[0]
    bm = _BLOCK_M
    grid = (n_tokens // bm,)

    gates, idxs, aux = pl.pallas_call(
        functools.partial(_router_block, n_tokens=n_tokens),
        grid=grid,
        in_specs=[
            pl.BlockSpec((bm, d_model), lambda i: (i, 0)),
            pl.BlockSpec((n_experts, d_model), lambda i: (0, 0)),
            pl.BlockSpec((n_experts, 1), lambda i: (0, 0)),
        ],
        out_specs=[
            pl.BlockSpec((bm, n_experts), lambda i: (i, 0)),
            pl.BlockSpec((bm, _TOP_K), lambda i: (i, 0)),
            pl.BlockSpec((1, 1), lambda i: (0, 0)),
        ],
        out_shape=[
            jax.ShapeDtypeStruct((n_tokens, n_experts), jnp.float32),
            jax.ShapeDtypeStruct((n_tokens, _TOP_K), jnp.int32),
            jax.ShapeDtypeStruct((1, 1), jnp.float32),
        ],
        scratch_shapes=[
            pltpu.VMEM((n_experts, 1), jnp.float32),
            pltpu.VMEM((n_experts, 1), jnp.float32),
        ],
    )(x, W, expert_bias.reshape(n_experts, 1))
    return gates, idxs, aux[0, 0]


# speedup vs baseline: 15.4219x; 1.1418x over previous
import functools

import jax
import jax.numpy as jnp
from jax.experimental import pallas as pl
from jax.experimental.pallas import tpu as pltpu

_D_MODEL = 768
_N_EXPERTS = 64
_TOP_K = 8
_AUX_COEF = 0.01
_BLOCK_M = 2048


def _router_block(x_ref, w_ref, b_ref, gates_ref, idx_ref, aux_ref,
                  f_acc, p_acc, *, n_tokens):
    i = pl.program_id(0)
    nsteps = pl.num_programs(0)

    logits = jax.lax.dot_general(
        w_ref[...], x_ref[...], (((1,), (1,)), ((), ())),
        preferred_element_type=jnp.float32,
    )
    biased = logits + b_ref[...]

    rev = jnp.float32(_N_EXPERTS) - jax.lax.broadcasted_iota(
        jnp.int32, logits.shape, 0).astype(jnp.float32)
    cur = biased
    sel_mask = jnp.zeros_like(logits)
    sel_idx_f = []
    for _ in range(_TOP_K):
        mx = jnp.max(cur, axis=0, keepdims=True)
        r = jnp.max(
            jnp.where(cur == mx, rev, 0.0), axis=0, keepdims=True)
        onehot = rev == r
        sel_mask += onehot.astype(jnp.float32)
        sel_idx_f.append(jnp.float32(_N_EXPERTS) - r)
        cur = jnp.where(onehot, -jnp.inf, cur)

    idxs = jnp.concatenate(sel_idx_f, axis=0).astype(jnp.int32)

    masked = jnp.where(sel_mask > 0, logits, -jnp.inf)
    m = jnp.max(masked, axis=0, keepdims=True)
    e = jnp.exp(masked - m)
    gates = e / jnp.sum(e, axis=0, keepdims=True)

    gates_ref[...] = gates.T
    idx_ref[...] = idxs.T

    f_part = jnp.sum((gates > 0).astype(jnp.float32), axis=1, keepdims=True)
    ml = jnp.max(logits, axis=0, keepdims=True)
    el = jnp.exp(logits - ml)
    p = el / jnp.sum(el, axis=0, keepdims=True)
    p_part = jnp.sum(p, axis=1, keepdims=True)

    @pl.when(i == 0)
    def _init():
        f_acc[...] = jnp.zeros_like(f_acc)
        p_acc[...] = jnp.zeros_like(p_acc)
        aux_ref[...] = jnp.zeros_like(aux_ref)

    f_acc[...] += f_part
    p_acc[...] += p_part

    @pl.when(i == nsteps - 1)
    def _finish():
        scale = _AUX_COEF * _N_EXPERTS / (float(n_tokens) * float(n_tokens))
        aux_ref[...] = (scale * jnp.sum(f_acc[...] * p_acc[...]))[None, None]


def kernel(x, W, expert_bias):
    n_tokens, d_model = x.shape
    n_experts = W.shape[0]
    bm = _BLOCK_M
    grid = (n_tokens // bm,)

    gates, idxs, aux = pl.pallas_call(
        functools.partial(_router_block, n_tokens=n_tokens),
        grid=grid,
        in_specs=[
            pl.BlockSpec((bm, d_model), lambda i: (i, 0)),
            pl.BlockSpec((n_experts, d_model), lambda i: (0, 0)),
            pl.BlockSpec((n_experts, 1), lambda i: (0, 0)),
        ],
        out_specs=[
            pl.BlockSpec((bm, n_experts), lambda i: (i, 0)),
            pl.BlockSpec((bm, _TOP_K), lambda i: (i, 0)),
            pl.BlockSpec((1, 1), lambda i: (0, 0)),
        ],
        out_shape=[
            jax.ShapeDtypeStruct((n_tokens, n_experts), jnp.float32),
            jax.ShapeDtypeStruct((n_tokens, _TOP_K), jnp.int32),
            jax.ShapeDtypeStruct((1, 1), jnp.float32),
        ],
        scratch_shapes=[
            pltpu.VMEM((n_experts, 1), jnp.float32),
            pltpu.VMEM((n_experts, 1), jnp.float32),
        ],
    )(x, W, expert_bias.reshape(n_experts, 1))
    return gates, idxs, aux[0, 0]


# speedup vs baseline: 16.4732x; 1.0682x over previous
import functools

import jax
import jax.numpy as jnp
from jax.experimental import pallas as pl
from jax.experimental.pallas import tpu as pltpu

_D_MODEL = 768
_N_EXPERTS = 64
_TOP_K = 8
_AUX_COEF = 0.01
_BLOCK_M = 4096


def _router_block(x_ref, w_ref, b_ref, gates_ref, idx_ref, aux_ref,
                  f_acc, p_acc, *, n_tokens):
    i = pl.program_id(0)
    nsteps = pl.num_programs(0)

    logits = jax.lax.dot_general(
        w_ref[...], x_ref[...], (((1,), (1,)), ((), ())),
        preferred_element_type=jnp.float32,
    )
    biased = logits + b_ref[...]

    rev = jnp.float32(_N_EXPERTS) - jax.lax.broadcasted_iota(
        jnp.int32, logits.shape, 0).astype(jnp.float32)
    cur = biased
    sel_mask = jnp.zeros_like(logits)
    sel_idx_f = []
    for _ in range(_TOP_K):
        mx = jnp.max(cur, axis=0, keepdims=True)
        r = jnp.max(
            jnp.where(cur == mx, rev, 0.0), axis=0, keepdims=True)
        onehot = rev == r
        sel_mask += onehot.astype(jnp.float32)
        sel_idx_f.append(jnp.float32(_N_EXPERTS) - r)
        cur = jnp.where(onehot, -jnp.inf, cur)

    idxs = jnp.concatenate(sel_idx_f, axis=0).astype(jnp.int32)

    masked = jnp.where(sel_mask > 0, logits, -jnp.inf)
    m = jnp.max(masked, axis=0, keepdims=True)
    e = jnp.exp(masked - m)
    gates = e / jnp.sum(e, axis=0, keepdims=True)

    gates_ref[...] = gates.T
    idx_ref[...] = idxs.T

    f_part = jnp.sum((gates > 0).astype(jnp.float32), axis=1, keepdims=True)
    ml = jnp.max(logits, axis=0, keepdims=True)
    el = jnp.exp(logits - ml)
    p = el / jnp.sum(el, axis=0, keepdims=True)
    p_part = jnp.sum(p, axis=1, keepdims=True)

    @pl.when(i == 0)
    def _init():
        f_acc[...] = jnp.zeros_like(f_acc)
        p_acc[...] = jnp.zeros_like(p_acc)
        aux_ref[...] = jnp.zeros_like(aux_ref)

    f_acc[...] += f_part
    p_acc[...] += p_part

    @pl.when(i == nsteps - 1)
    def _finish():
        scale = _AUX_COEF * _N_EXPERTS / (float(n_tokens) * float(n_tokens))
        aux_ref[...] = (scale * jnp.sum(f_acc[...] * p_acc[...]))[None, None]


def kernel(x, W, expert_bias):
    n_tokens, d_model = x.shape
    n_experts = W.shape[0]
    bm = _BLOCK_M
    grid = (n_tokens // bm,)

    gates, idxs, aux = pl.pallas_call(
        functools.partial(_router_block, n_tokens=n_tokens),
        grid=grid,
        in_specs=[
            pl.BlockSpec((bm, d_model), lambda i: (i, 0)),
            pl.BlockSpec((n_experts, d_model), lambda i: (0, 0)),
            pl.BlockSpec((n_experts, 1), lambda i: (0, 0)),
        ],
        out_specs=[
            pl.BlockSpec((bm, n_experts), lambda i: (i, 0)),
            pl.BlockSpec((bm, _TOP_K), lambda i: (i, 0)),
            pl.BlockSpec((1, 1), lambda i: (0, 0)),
        ],
        out_shape=[
            jax.ShapeDtypeStruct((n_tokens, n_experts), jnp.float32),
            jax.ShapeDtypeStruct((n_tokens, _TOP_K), jnp.int32),
            jax.ShapeDtypeStruct((1, 1), jnp.float32),
        ],
        scratch_shapes=[
            pltpu.VMEM((n_experts, 1), jnp.float32),
            pltpu.VMEM((n_experts, 1), jnp.float32),
        ],
    )(x, W, expert_bias.reshape(n_experts, 1))
    return gates, idxs, aux[0, 0]


# speedup vs baseline: 16.9205x; 1.0272x over previous
import functools

import jax
import jax.numpy as jnp
from jax.experimental import pallas as pl
from jax.experimental.pallas import tpu as pltpu

_D_MODEL = 768
_N_EXPERTS = 64
_TOP_K = 8
_AUX_COEF = 0.01
_BLOCK_M = 4096


def _router_block(x_ref, w_ref, b_ref, gates_ref, idx_ref, aux_ref,
                  f_acc, p_acc, *, n_tokens):
    i = pl.program_id(0)
    nsteps = pl.num_programs(0)

    logits = jax.lax.dot_general(
        w_ref[...], x_ref[...], (((1,), (1,)), ((), ())),
        preferred_element_type=jnp.float32,
    )
    biased = logits + b_ref[...]

    rev = _N_EXPERTS - jax.lax.broadcasted_iota(jnp.int32, logits.shape, 0)
    cur = biased
    sel_r = []
    for _ in range(_TOP_K):
        mx = jnp.max(cur, axis=0, keepdims=True)
        r = jnp.max(
            jnp.where(cur == mx, rev, 0), axis=0, keepdims=True)
        sel_r.append(r)
        cur = jnp.where(rev == r, -jnp.inf, cur)

    idxs = _N_EXPERTS - jnp.concatenate(sel_r, axis=0)

    masked = jnp.where(cur == -jnp.inf, logits, -jnp.inf)
    m = jnp.max(masked, axis=0, keepdims=True)
    e = jnp.exp(masked - m)
    gates = e / jnp.sum(e, axis=0, keepdims=True)

    gates_ref[...] = gates.T
    idx_ref[...] = idxs.T

    f_part = jnp.sum((gates > 0).astype(jnp.float32), axis=1, keepdims=True)
    ml = jnp.max(logits, axis=0, keepdims=True)
    el = jnp.exp(logits - ml)
    p = el / jnp.sum(el, axis=0, keepdims=True)
    p_part = jnp.sum(p, axis=1, keepdims=True)

    @pl.when(i == 0)
    def _init():
        f_acc[...] = jnp.zeros_like(f_acc)
        p_acc[...] = jnp.zeros_like(p_acc)
        aux_ref[...] = jnp.zeros_like(aux_ref)

    f_acc[...] += f_part
    p_acc[...] += p_part

    @pl.when(i == nsteps - 1)
    def _finish():
        scale = _AUX_COEF * _N_EXPERTS / (float(n_tokens) * float(n_tokens))
        aux_ref[...] = (scale * jnp.sum(f_acc[...] * p_acc[...]))[None, None]


def kernel(x, W, expert_bias):
    n_tokens, d_model = x.shape
    n_experts = W.shape[0]
    bm = _BLOCK_M
    grid = (n_tokens // bm,)

    gates, idxs, aux = pl.pallas_call(
        functools.partial(_router_block, n_tokens=n_tokens),
        grid=grid,
        in_specs=[
            pl.BlockSpec((bm, d_model), lambda i: (i, 0)),
            pl.BlockSpec((n_experts, d_model), lambda i: (0, 0)),
            pl.BlockSpec((n_experts, 1), lambda i: (0, 0)),
        ],
        out_specs=[
            pl.BlockSpec((bm, n_experts), lambda i: (i, 0)),
            pl.BlockSpec((bm, _TOP_K), lambda i: (i, 0)),
            pl.BlockSpec((1, 1), lambda i: (0, 0)),
        ],
        out_shape=[
            jax.ShapeDtypeStruct((n_tokens, n_experts), jnp.float32),
            jax.ShapeDtypeStruct((n_tokens, _TOP_K), jnp.int32),
            jax.ShapeDtypeStruct((1, 1), jnp.float32),
        ],
        scratch_shapes=[
            pltpu.VMEM((n_experts, 1), jnp.float32),
            pltpu.VMEM((n_experts, 1), jnp.float32),
        ],
    )(x, W, expert_bias.reshape(n_experts, 1))
    return gates, idxs, aux[0, 0]


# speedup vs baseline: 16.9311x; 1.0006x over previous
import functools

import jax
import jax.numpy as jnp
from jax.experimental import pallas as pl
from jax.experimental.pallas import tpu as pltpu

_D_MODEL = 768
_N_EXPERTS = 64
_TOP_K = 8
_AUX_COEF = 0.01
_BLOCK_M = 4096
_N_CHUNKS = 8


def _router_block(x_ref, w_ref, b_ref, gates_ref, idx_ref, aux_ref,
                  f_acc, p_acc, *, n_tokens):
    i = pl.program_id(0)
    nsteps = pl.num_programs(0)

    logits = jax.lax.dot_general(
        w_ref[...], x_ref[...], (((1,), (1,)), ((), ())),
        preferred_element_type=jnp.float32,
    )
    bias = b_ref[...]

    bm = logits.shape[1]
    bc = bm // _N_CHUNKS
    rev = _N_EXPERTS - jax.lax.broadcasted_iota(
        jnp.int32, (_N_EXPERTS, bc), 0)

    f_part = jnp.zeros((_N_EXPERTS, 1), jnp.float32)
    p_part = jnp.zeros((_N_EXPERTS, 1), jnp.float32)
    for c in range(_N_CHUNKS):
        lg = logits[:, c * bc:(c + 1) * bc]
        cur = lg + bias
        sel_r = []
        for _ in range(_TOP_K):
            mx = jnp.max(cur, axis=0, keepdims=True)
            r = jnp.max(
                jnp.where(cur == mx, rev, 0), axis=0, keepdims=True)
            sel_r.append(r)
            cur = jnp.where(rev == r, -jnp.inf, cur)

        idxs = _N_EXPERTS - jnp.concatenate(sel_r, axis=0)

        masked = jnp.where(cur == -jnp.inf, lg, -jnp.inf)
        m = jnp.max(masked, axis=0, keepdims=True)
        e = jnp.exp(masked - m)
        gates = e / jnp.sum(e, axis=0, keepdims=True)

        gates_ref[c * bc:(c + 1) * bc, :] = gates.T
        idx_ref[c * bc:(c + 1) * bc, :] = idxs.T

        f_part += jnp.sum(
            (gates > 0).astype(jnp.float32), axis=1, keepdims=True)
        ml = jnp.max(lg, axis=0, keepdims=True)
        el = jnp.exp(lg - ml)
        p = el / jnp.sum(el, axis=0, keepdims=True)
        p_part += jnp.sum(p, axis=1, keepdims=True)

    @pl.when(i == 0)
    def _init():
        f_acc[...] = jnp.zeros_like(f_acc)
        p_acc[...] = jnp.zeros_like(p_acc)
        aux_ref[...] = jnp.zeros_like(aux_ref)

    f_acc[...] += f_part
    p_acc[...] += p_part

    @pl.when(i == nsteps - 1)
    def _finish():
        scale = _AUX_COEF * _N_EXPERTS / (float(n_tokens) * float(n_tokens))
        aux_ref[...] = (scale * jnp.sum(f_acc[...] * p_acc[...]))[None, None]


def kernel(x, W, expert_bias):
    n_tokens, d_model = x.shape
    n_experts = W.shape[0]
    bm = _BLOCK_M
    grid = (n_tokens // bm,)

    gates, idxs, aux = pl.pallas_call(
        functools.partial(_router_block, n_tokens=n_tokens),
        grid=grid,
        in_specs=[
            pl.BlockSpec((bm, d_model), lambda i: (i, 0)),
            pl.BlockSpec((n_experts, d_model), lambda i: (0, 0)),
            pl.BlockSpec((n_experts, 1), lambda i: (0, 0)),
        ],
        out_specs=[
            pl.BlockSpec((bm, n_experts), lambda i: (i, 0)),
            pl.BlockSpec((bm, _TOP_K), lambda i: (i, 0)),
            pl.BlockSpec((1, 1), lambda i: (0, 0)),
        ],
        out_shape=[
            jax.ShapeDtypeStruct((n_tokens, n_experts), jnp.float32),
            jax.ShapeDtypeStruct((n_tokens, _TOP_K), jnp.int32),
            jax.ShapeDtypeStruct((1, 1), jnp.float32),
        ],
        scratch_shapes=[
            pltpu.VMEM((n_experts, 1), jnp.float32),
            pltpu.VMEM((n_experts, 1), jnp.float32),
        ],
    )(x, W, expert_bias.reshape(n_experts, 1))
    return gates, idxs, aux[0, 0]
